# baseline (speedup 1.0000x reference)
import functools

import jax
import jax.numpy as jnp
from jax import lax
from jax.experimental import pallas as pl
from jax.experimental.pallas import tpu as pltpu
from jax.experimental.pallas import tpu_sc as plsc

N_NODES = 10000
D = 128
NC = 2
NS = 16
NW = NC * NS
N_PAD = 10240


def _sc_degree(dst3):
    _, M, C = dst3.shape
    rpt = N_PAD // NS

    mesh = plsc.VectorSubcoreMesh(core_axis_name="c", subcore_axis_name="s")

    @functools.partial(
        pl.kernel,
        out_type=jax.ShapeDtypeStruct((NC, N_PAD), jnp.float32),
        mesh=mesh,
        scratch_types=[
            pltpu.VMEM((M, C), jnp.int32),
            pltpu.VMEM((C,), jnp.float32),
            pltpu.VMEM((rpt,), jnp.float32),
            pltpu.VMEM_SHARED((N_PAD,), jnp.float32),
        ],
    )
    def deg_kernel(dst_hbm, out_hbm, idx_v, ones_v, zb_v, hist_s):
        cid = lax.axis_index("c")
        sid = lax.axis_index("s")
        wid = sid * NC + cid

        def fill_ones(i, _):
            ones_v[pl.ds(i * 16, 16)] = jnp.ones((16,), jnp.float32)
            return 0

        lax.fori_loop(0, C // 16, fill_ones, 0)

        def fill_zeros(i, _):
            zb_v[pl.ds(i * 16, 16)] = jnp.zeros((16,), jnp.float32)
            return 0

        lax.fori_loop(0, rpt // 16, fill_zeros, 0)

        pltpu.sync_copy(dst_hbm.at[wid], idx_v)
        pltpu.sync_copy(zb_v, hist_s.at[pl.ds(sid * rpt, rpt)])
        plsc.subcore_barrier()

        def scatter_chunk(j, _):
            pltpu.sync_copy(ones_v, hist_s.at[idx_v.at[j]], add=True)
            return 0

        lax.fori_loop(0, M, scatter_chunk, 0)
        plsc.subcore_barrier()
        pltpu.sync_copy(hist_s.at[pl.ds(sid * rpt, rpt)],
                        out_hbm.at[cid, pl.ds(sid * rpt, rpt)])

    return deg_kernel(dst3)


def _sc_scatter(g, src4, dst4):
    _, NSTAGE, MB, C = src4.shape
    NB = 2
    rpt = N_PAD // NS
    zrows = 32

    mesh = plsc.VectorSubcoreMesh(core_axis_name="c", subcore_axis_name="s")

    @functools.partial(
        pl.kernel,
        out_type=jax.ShapeDtypeStruct((NC, N_PAD, D), jnp.float32),
        mesh=mesh,
        scratch_types=[
            pltpu.VMEM((MB, C), jnp.int32),
            pltpu.VMEM((MB, C), jnp.int32),
            pltpu.VMEM((NB, C, D), jnp.float32),
            pltpu.VMEM((zrows, D), jnp.float32),
            pltpu.VMEM_SHARED((N_PAD, D), jnp.float32),
            [pltpu.SemaphoreType.DMA] * NB,
            [pltpu.SemaphoreType.DMA] * NB,
        ],
    )
    def scat_kernel(g_hbm, src_hbm, dst_hbm, out_hbm,
                    sidx_v, didx_v, buf_v, zb_v, acc_s, semg, sems):
        cid = lax.axis_index("c")
        sid = lax.axis_index("s")
        wid = sid * NC + cid

        def fill_zeros(i, _):
            for k in range(D // 16):
                zb_v[i, pl.ds(k * 16, 16)] = jnp.zeros((16,), jnp.float32)
            return 0

        lax.fori_loop(0, zrows, fill_zeros, 0)

        def zero_acc(r, _):
            pltpu.sync_copy(zb_v, acc_s.at[pl.ds(sid * rpt + r * zrows, zrows)])
            return 0

        lax.fori_loop(0, rpt // zrows, zero_acc, 0)
        plsc.subcore_barrier()

        def fire_g(j, q):
            pltpu.async_copy(g_hbm.at[sidx_v.at[j]], buf_v.at[q], semg[q])

        def wait_g(q):
            pltpu.make_async_copy(g_hbm.at[sidx_v.at[0]], buf_v.at[q],
                                  semg[q]).wait()

        def fire_s(j, q):
            pltpu.async_copy(buf_v.at[q], acc_s.at[didx_v.at[j]], sems[q],
                             add=True)

        def wait_s(q):
            pltpu.make_async_copy(buf_v.at[q], acc_s.at[didx_v.at[0]],
                                  sems[q]).wait()

        def stage(r, _):
            pltpu.sync_copy(src_hbm.at[wid, r], sidx_v)
            pltpu.sync_copy(dst_hbm.at[wid, r], didx_v)
            fire_g(0, 0)

            def pair(p, _):
                for q in range(NB):
                    j = NB * p + q
                    wait_g(q)
                    fire_s(j, q)

                    @pl.when(j + 1 < MB)
                    def _():
                        @pl.when(j >= 1)
                        def _():
                            wait_s(1 - q)

                        fire_g(j + 1, 1 - q)
                return 0

            lax.fori_loop(0, MB // NB, pair, 0)
            wait_s(0)
            wait_s(1)
            return 0

        lax.fori_loop(0, NSTAGE, stage, 0)
        plsc.subcore_barrier()
        pltpu.sync_copy(acc_s.at[pl.ds(sid * rpt, rpt)],
                        out_hbm.at[cid, pl.ds(sid * rpt, rpt)])

    return scat_kernel(g, src4, dst4)


def _tc_transform(x, W, deg_b):
    n = x.shape[0]
    blk = 1000

    def body(x_ref, w_ref, d_ref, g_ref):
        h = jnp.dot(x_ref[...], w_ref[...], preferred_element_type=jnp.float32)
        g_ref[...] = h * lax.rsqrt(d_ref[...])

    return pl.pallas_call(
        body,
        grid=(n // blk,),
        in_specs=[
            pl.BlockSpec((blk, D), lambda i: (i, 0)),
            pl.BlockSpec((D, D), lambda i: (0, 0)),
            pl.BlockSpec((blk, D), lambda i: (i, 0)),
        ],
        out_specs=pl.BlockSpec((blk, D), lambda i: (i, 0)),
        out_shape=jax.ShapeDtypeStruct((n, D), jnp.float32),
    )(x, W, deg_b)


def _tc_final(deg_b, acc, g, b):
    n = g.shape[0]
    blk = 1000

    def body(d_ref, a_ref, g_ref, b_ref, o_ref):
        dinv = lax.rsqrt(d_ref[...])
        o_ref[...] = dinv * (a_ref[0] + a_ref[1] + g_ref[...]) + b_ref[...]

    return pl.pallas_call(
        body,
        grid=(n // blk,),
        in_specs=[
            pl.BlockSpec((blk, D), lambda i: (i, 0)),
            pl.BlockSpec((NC, blk, D), lambda i: (0, i, 0)),
            pl.BlockSpec((blk, D), lambda i: (i, 0)),
            pl.BlockSpec((1, D), lambda i: (0, 0)),
        ],
        out_specs=pl.BlockSpec((blk, D), lambda i: (i, 0)),
        out_shape=jax.ShapeDtypeStruct((n, D), jnp.float32),
    )(deg_b, acc, g, b)


def kernel(x, edge_index, node_type, edge_type, W, b):
    n, d = x.shape
    e = edge_index.shape[1]
    ept = e // NW
    C = 80
    M = ept // C
    CM = 100
    MB = 20
    NSTAGE = ept // (MB * CM)

    src = edge_index[0].astype(jnp.int32)
    dst = edge_index[1].astype(jnp.int32)
    dst3 = dst.reshape(NW, M, C)
    src4 = src.reshape(NW, NSTAGE, MB, CM)
    dst4 = dst.reshape(NW, NSTAGE, MB, CM)

    degp = _sc_degree(dst3)
    deg = degp[0, :n] + degp[1, :n] + 1.0
    deg_b = jnp.broadcast_to(deg[:, None], (n, d))

    g = _tc_transform(x, W, deg_b)
    acc = _sc_scatter(g, src4, dst4)
    out = _tc_final(deg_b, acc, g, b[None, :])
    return out


# speedup vs baseline: 1.0222x; 1.0222x over previous
import functools

import jax
import jax.numpy as jnp
from jax import lax
from jax.experimental import pallas as pl
from jax.experimental.pallas import tpu as pltpu
from jax.experimental.pallas import tpu_sc as plsc

N_NODES = 10000
D = 128
NC = 2
NS = 16
NW = NC * NS
N_PAD = 10240


def _sc_degree(dst3):
    _, M, C = dst3.shape
    rpt = N_PAD // NS

    mesh = plsc.VectorSubcoreMesh(core_axis_name="c", subcore_axis_name="s")

    @functools.partial(
        pl.kernel,
        out_type=jax.ShapeDtypeStruct((NC, N_PAD), jnp.float32),
        mesh=mesh,
        scratch_types=[
            pltpu.VMEM((M, C), jnp.int32),
            pltpu.VMEM((C,), jnp.float32),
            pltpu.VMEM((rpt,), jnp.float32),
            pltpu.VMEM_SHARED((N_PAD,), jnp.float32),
        ],
    )
    def deg_kernel(dst_hbm, out_hbm, idx_v, ones_v, zb_v, hist_s):
        cid = lax.axis_index("c")
        sid = lax.axis_index("s")
        wid = sid * NC + cid

        def fill_ones(i, _):
            ones_v[pl.ds(i * 16, 16)] = jnp.ones((16,), jnp.float32)
            return 0

        lax.fori_loop(0, C // 16, fill_ones, 0)

        def fill_zeros(i, _):
            zb_v[pl.ds(i * 16, 16)] = jnp.zeros((16,), jnp.float32)
            return 0

        lax.fori_loop(0, rpt // 16, fill_zeros, 0)

        pltpu.sync_copy(dst_hbm.at[wid], idx_v)
        pltpu.sync_copy(zb_v, hist_s.at[pl.ds(sid * rpt, rpt)])
        plsc.subcore_barrier()

        def scatter_chunk(j, _):
            pltpu.sync_copy(ones_v, hist_s.at[idx_v.at[j]], add=True)
            return 0

        lax.fori_loop(0, M, scatter_chunk, 0)
        plsc.subcore_barrier()
        pltpu.sync_copy(hist_s.at[pl.ds(sid * rpt, rpt)],
                        out_hbm.at[cid, pl.ds(sid * rpt, rpt)])

    return deg_kernel(dst3)


def _sc_scatter(g, src4, dst4):
    _, NSTAGE, MB, C = src4.shape
    NB = 2
    rpt = N_PAD // NS
    zrows = 32

    mesh = plsc.VectorSubcoreMesh(core_axis_name="c", subcore_axis_name="s")

    @functools.partial(
        pl.kernel,
        out_type=jax.ShapeDtypeStruct((NC, N_PAD, D), jnp.float32),
        mesh=mesh,
        scratch_types=[
            pltpu.VMEM((MB, C), jnp.int32),
            pltpu.VMEM((MB, C), jnp.int32),
            pltpu.VMEM((NB, C, D), jnp.float32),
            pltpu.VMEM((zrows, D), jnp.float32),
            pltpu.VMEM_SHARED((N_PAD, D), jnp.float32),
            [pltpu.SemaphoreType.DMA] * NB,
            [pltpu.SemaphoreType.DMA] * NB,
        ],
    )
    def scat_kernel(g_hbm, src_hbm, dst_hbm, out_hbm,
                    sidx_v, didx_v, buf_v, zb_v, acc_s, semg, sems):
        cid = lax.axis_index("c")
        sid = lax.axis_index("s")
        wid = sid * NC + cid

        for i in range(zrows):
            for k in range(D // 16):
                zb_v[i, pl.ds(k * 16, 16)] = jnp.zeros((16,), jnp.float32)

        def zero_acc(r, _):
            pltpu.sync_copy(zb_v, acc_s.at[pl.ds(sid * rpt + r * zrows, zrows)])
            return 0

        lax.fori_loop(0, rpt // zrows, zero_acc, 0)
        plsc.subcore_barrier()

        def fire_g(j, q):
            pltpu.async_copy(g_hbm.at[sidx_v.at[j]], buf_v.at[q], semg[q])

        def wait_g(q):
            pltpu.make_async_copy(g_hbm.at[sidx_v.at[0]], buf_v.at[q],
                                  semg[q]).wait()

        def fire_s(j, q):
            pltpu.async_copy(buf_v.at[q], acc_s.at[didx_v.at[j]], sems[q],
                             add=True)

        def wait_s(q):
            pltpu.make_async_copy(buf_v.at[q], acc_s.at[didx_v.at[0]],
                                  sems[q]).wait()

        def stage(r, _):
            pltpu.sync_copy(src_hbm.at[wid, r], sidx_v)
            pltpu.sync_copy(dst_hbm.at[wid, r], didx_v)
            fire_g(0, 0)

            def pair(p, _):
                for q in range(NB):
                    j = NB * p + q
                    wait_g(q)
                    fire_s(j, q)

                    @pl.when(j + 1 < MB)
                    def _():
                        @pl.when(j >= 1)
                        def _():
                            wait_s(1 - q)

                        fire_g(j + 1, 1 - q)
                return 0

            lax.fori_loop(0, MB // NB, pair, 0)
            wait_s(0)
            wait_s(1)
            return 0

        lax.fori_loop(0, NSTAGE, stage, 0)
        plsc.subcore_barrier()
        pltpu.sync_copy(acc_s.at[pl.ds(sid * rpt, rpt)],
                        out_hbm.at[cid, pl.ds(sid * rpt, rpt)])

    return scat_kernel(g, src4, dst4)


def _tc_matmul(x, W):
    n = x.shape[0]
    blk = 2000

    def body(x_ref, w_ref, h_ref):
        h_ref[...] = jnp.dot(x_ref[...], w_ref[...],
                             preferred_element_type=jnp.float32)

    return pl.pallas_call(
        body,
        grid=(n // blk,),
        in_specs=[
            pl.BlockSpec((blk, D), lambda i: (i, 0)),
            pl.BlockSpec((D, D), lambda i: (0, 0)),
        ],
        out_specs=pl.BlockSpec((blk, D), lambda i: (i, 0)),
        out_shape=jax.ShapeDtypeStruct((n, D), jnp.float32),
    )(x, W)


def _tc_scale(h, deg_b):
    n = h.shape[0]
    blk = 2000

    def body(h_ref, d_ref, g_ref):
        g_ref[...] = h_ref[...] * lax.rsqrt(d_ref[...])

    return pl.pallas_call(
        body,
        grid=(n // blk,),
        in_specs=[
            pl.BlockSpec((blk, D), lambda i: (i, 0)),
            pl.BlockSpec((blk, D), lambda i: (i, 0)),
        ],
        out_specs=pl.BlockSpec((blk, D), lambda i: (i, 0)),
        out_shape=jax.ShapeDtypeStruct((n, D), jnp.float32),
    )(h, deg_b)


def _tc_final(deg_b, acc, g, b):
    n = g.shape[0]
    blk = 2000

    def body(d_ref, a_ref, g_ref, b_ref, o_ref):
        dinv = lax.rsqrt(d_ref[...])
        o_ref[...] = dinv * (a_ref[0] + a_ref[1] + g_ref[...]) + b_ref[...]

    return pl.pallas_call(
        body,
        grid=(n // blk,),
        in_specs=[
            pl.BlockSpec((blk, D), lambda i: (i, 0)),
            pl.BlockSpec((NC, blk, D), lambda i: (0, i, 0)),
            pl.BlockSpec((blk, D), lambda i: (i, 0)),
            pl.BlockSpec((1, D), lambda i: (0, 0)),
        ],
        out_specs=pl.BlockSpec((blk, D), lambda i: (i, 0)),
        out_shape=jax.ShapeDtypeStruct((n, D), jnp.float32),
    )(deg_b, acc, g, b)


def kernel(x, edge_index, node_type, edge_type, W, b):
    n, d = x.shape
    e = edge_index.shape[1]
    ept = e // NW
    C = 80
    M = ept // C
    CM = 100
    MB = 20
    NSTAGE = ept // (MB * CM)

    src = edge_index[0].astype(jnp.int32)
    dst = edge_index[1].astype(jnp.int32)
    dst3 = dst.reshape(NW, M, C)
    src4 = src.reshape(NW, NSTAGE, MB, CM)
    dst4 = dst.reshape(NW, NSTAGE, MB, CM)

    h = _tc_matmul(x, W)
    degp = _sc_degree(dst3)
    deg = degp[0, :n] + degp[1, :n] + 1.0
    deg_b = jnp.broadcast_to(deg[:, None], (n, d))

    g = _tc_scale(h, deg_b)
    acc = _sc_scatter(g, src4, dst4)
    out = _tc_final(deg_b, acc, g, b[None, :])
    return out


# speedup vs baseline: 1.0422x; 1.0195x over previous
import functools

import jax
import jax.numpy as jnp
from jax import lax
from jax.experimental import pallas as pl
from jax.experimental.pallas import tpu as pltpu
from jax.experimental.pallas import tpu_sc as plsc

N_NODES = 10000
D = 128
NC = 2
NS = 16
NW = NC * NS
N_PAD = 10240


def _sc_degree(dst3):
    _, M, C = dst3.shape
    rpt = N_PAD // NS

    mesh = plsc.VectorSubcoreMesh(core_axis_name="c", subcore_axis_name="s")

    @functools.partial(
        pl.kernel,
        out_type=jax.ShapeDtypeStruct((NC, N_PAD), jnp.float32),
        mesh=mesh,
        scratch_types=[
            pltpu.VMEM((M, C), jnp.int32),
            pltpu.VMEM((C,), jnp.float32),
            pltpu.VMEM((rpt,), jnp.float32),
            pltpu.VMEM_SHARED((N_PAD,), jnp.float32),
        ],
    )
    def deg_kernel(dst_hbm, out_hbm, idx_v, ones_v, zb_v, hist_s):
        cid = lax.axis_index("c")
        sid = lax.axis_index("s")
        wid = sid * NC + cid

        def fill_ones(i, _):
            ones_v[pl.ds(i * 16, 16)] = jnp.ones((16,), jnp.float32)
            return 0

        lax.fori_loop(0, C // 16, fill_ones, 0)

        def fill_zeros(i, _):
            zb_v[pl.ds(i * 16, 16)] = jnp.zeros((16,), jnp.float32)
            return 0

        lax.fori_loop(0, rpt // 16, fill_zeros, 0)

        pltpu.sync_copy(dst_hbm.at[wid], idx_v)
        pltpu.sync_copy(zb_v, hist_s.at[pl.ds(sid * rpt, rpt)])
        plsc.subcore_barrier()

        def scatter_chunk(j, _):
            pltpu.sync_copy(ones_v, hist_s.at[idx_v.at[j]], add=True)
            return 0

        lax.fori_loop(0, M, scatter_chunk, 0)
        plsc.subcore_barrier()
        pltpu.sync_copy(hist_s.at[pl.ds(sid * rpt, rpt)],
                        out_hbm.at[cid, pl.ds(sid * rpt, rpt)])

    return deg_kernel(dst3)


def _sc_scatter(g, src4, dst4):
    _, NSTAGE, MB, C = src4.shape
    NB = 2
    rpt = N_PAD // NS
    zrows = 16

    mesh = plsc.VectorSubcoreMesh(core_axis_name="c", subcore_axis_name="s")

    @functools.partial(
        pl.kernel,
        out_type=jax.ShapeDtypeStruct((NC, N_PAD, D), jnp.float32),
        mesh=mesh,
        scratch_types=[
            pltpu.VMEM((MB, C), jnp.int32),
            pltpu.VMEM((MB, C), jnp.int32),
            pltpu.VMEM((NB, C, D), jnp.float32),
            pltpu.VMEM((zrows, D), jnp.float32),
            pltpu.VMEM_SHARED((N_PAD, D), jnp.float32),
            [pltpu.SemaphoreType.DMA] * NB,
            [pltpu.SemaphoreType.DMA] * NB,
        ],
    )
    def scat_kernel(g_hbm, src_hbm, dst_hbm, out_hbm,
                    sidx_v, didx_v, buf_v, zb_v, acc_s, semg, sems):
        cid = lax.axis_index("c")
        sid = lax.axis_index("s")
        wid = sid * NC + cid

        for i in range(zrows):
            for k in range(D // 16):
                zb_v[i, pl.ds(k * 16, 16)] = jnp.zeros((16,), jnp.float32)

        def zero_acc(r, _):
            pltpu.sync_copy(zb_v, acc_s.at[pl.ds(sid * rpt + r * zrows, zrows)])
            return 0

        lax.fori_loop(0, rpt // zrows, zero_acc, 0)
        plsc.subcore_barrier()

        def fire_g(j, q):
            pltpu.async_copy(g_hbm.at[sidx_v.at[j]], buf_v.at[q], semg[q])

        def wait_g(q):
            pltpu.make_async_copy(g_hbm.at[sidx_v.at[0]], buf_v.at[q],
                                  semg[q]).wait()

        def fire_s(j, q):
            pltpu.async_copy(buf_v.at[q], acc_s.at[didx_v.at[j]], sems[q],
                             add=True)

        def wait_s(q):
            pltpu.make_async_copy(buf_v.at[q], acc_s.at[didx_v.at[0]],
                                  sems[q]).wait()

        def stage(r, _):
            pltpu.sync_copy(src_hbm.at[wid, r], sidx_v)
            pltpu.sync_copy(dst_hbm.at[wid, r], didx_v)
            fire_g(0, 0)

            def pair(p, _):
                for q in range(NB):
                    j = NB * p + q
                    wait_g(q)
                    fire_s(j, q)

                    @pl.when(j + 1 < MB)
                    def _():
                        @pl.when(j >= 1)
                        def _():
                            wait_s(1 - q)

                        fire_g(j + 1, 1 - q)
                return 0

            lax.fori_loop(0, MB // NB, pair, 0)
            wait_s(0)
            wait_s(1)
            return 0

        lax.fori_loop(0, NSTAGE, stage, 0)
        plsc.subcore_barrier()
        pltpu.sync_copy(acc_s.at[pl.ds(sid * rpt, rpt)],
                        out_hbm.at[cid, pl.ds(sid * rpt, rpt)])

    return scat_kernel(g, src4, dst4)


def _tc_matmul(x, W):
    n = x.shape[0]
    blk = 2000

    def body(x_ref, w_ref, h_ref):
        h_ref[...] = jnp.dot(x_ref[...], w_ref[...],
                             preferred_element_type=jnp.float32)

    return pl.pallas_call(
        body,
        grid=(n // blk,),
        in_specs=[
            pl.BlockSpec((blk, D), lambda i: (i, 0)),
            pl.BlockSpec((D, D), lambda i: (0, 0)),
        ],
        out_specs=pl.BlockSpec((blk, D), lambda i: (i, 0)),
        out_shape=jax.ShapeDtypeStruct((n, D), jnp.float32),
    )(x, W)


def _tc_scale(h, deg_b):
    n = h.shape[0]
    blk = 2000

    def body(h_ref, d_ref, g_ref):
        g_ref[...] = h_ref[...] * lax.rsqrt(d_ref[...])

    return pl.pallas_call(
        body,
        grid=(n // blk,),
        in_specs=[
            pl.BlockSpec((blk, D), lambda i: (i, 0)),
            pl.BlockSpec((blk, D), lambda i: (i, 0)),
        ],
        out_specs=pl.BlockSpec((blk, D), lambda i: (i, 0)),
        out_shape=jax.ShapeDtypeStruct((n, D), jnp.float32),
    )(h, deg_b)


def _tc_final(deg_b, acc, g, b):
    n = g.shape[0]
    blk = 2000

    def body(d_ref, a_ref, g_ref, b_ref, o_ref):
        dinv = lax.rsqrt(d_ref[...])
        o_ref[...] = dinv * (a_ref[0] + a_ref[1] + g_ref[...]) + b_ref[...]

    return pl.pallas_call(
        body,
        grid=(n // blk,),
        in_specs=[
            pl.BlockSpec((blk, D), lambda i: (i, 0)),
            pl.BlockSpec((NC, blk, D), lambda i: (0, i, 0)),
            pl.BlockSpec((blk, D), lambda i: (i, 0)),
            pl.BlockSpec((1, D), lambda i: (0, 0)),
        ],
        out_specs=pl.BlockSpec((blk, D), lambda i: (i, 0)),
        out_shape=jax.ShapeDtypeStruct((n, D), jnp.float32),
    )(deg_b, acc, g, b)


def kernel(x, edge_index, node_type, edge_type, W, b):
    n, d = x.shape
    e = edge_index.shape[1]
    ept = e // NW
    C = 80
    M = ept // C
    CM = 125
    MB = 8
    NSTAGE = ept // (MB * CM)

    src = edge_index[0].astype(jnp.int32)
    dst = edge_index[1].astype(jnp.int32)
    dst3 = dst.reshape(NW, M, C)
    src4 = src.reshape(NW, NSTAGE, MB, CM)
    dst4 = dst.reshape(NW, NSTAGE, MB, CM)

    h = _tc_matmul(x, W)
    degp = _sc_degree(dst3)
    deg = degp[0, :n] + degp[1, :n] + 1.0
    deg_b = jnp.broadcast_to(deg[:, None], (n, d))

    g = _tc_scale(h, deg_b)
    acc = _sc_scatter(g, src4, dst4)
    out = _tc_final(deg_b, acc, g, b[None, :])
    return out
